# baseline (speedup 1.0000x reference)
import functools

import jax
import jax.numpy as jnp
import numpy as np
from jax.experimental import pallas as pl
from jax.experimental.pallas import tpu as pltpu

_TOP_N_PRE = 6000
_TOP_N_POST = 1000
_THRESH = 0.7
_FEATURE_SHAPE = (100, 50)
_FEATURE_STRIDE = 16
_ANCHOR_SIZES = (64.0, 128.0, 256.0, 512.0)
_N = _FEATURE_SHAPE[0] * _FEATURE_SHAPE[1] * len(_ANCHOR_SIZES)
_BL = 128
_NROW = 157
_NPAD = _NROW * _BL
_C = 2048
_CROWS = _C // _BL
_S_FULL = 6016
_OUTPAD = 1128

_HIGHEST = jax.lax.Precision.HIGHEST


def _anchors_t():
    H, W = _FEATURE_SHAPE
    shift_x = (np.arange(W, dtype=np.float32) + 0.5) * _FEATURE_STRIDE
    shift_y = (np.arange(H, dtype=np.float32) + 0.5) * _FEATURE_STRIDE
    yy, xx = np.meshgrid(shift_y, shift_x, indexing="ij")
    ctr = np.stack([xx.ravel(), yy.ravel()], axis=1)
    ws = np.asarray(_ANCHOR_SIZES, np.float32)
    hs = np.asarray(_ANCHOR_SIZES, np.float32)
    wh = np.stack([ws, hs], axis=1)
    lo = ctr[:, None, :] - wh[None, :, :] / 2.0
    hi = ctr[:, None, :] + wh[None, :, :] / 2.0
    boxes = np.concatenate([lo, hi], axis=-1).reshape(-1, 4)
    return jnp.asarray(boxes.T)


def _decode_body(anc_ref, reg_ref, info_ref, out_ref):
    ax1 = anc_ref[0, :]
    ay1 = anc_ref[1, :]
    ax2 = anc_ref[2, :]
    ay2 = anc_ref[3, :]
    aw = ax2 - ax1
    ah = ay2 - ay1
    acx = ax1 + aw * 0.5
    acy = ay1 + ah * 0.5
    dx = reg_ref[0, 0, :]
    dy = reg_ref[0, 1, :]
    dw = reg_ref[0, 2, :]
    dh = reg_ref[0, 3, :]
    cx = acx + dx * aw
    cy = acy + dy * ah
    w = aw * jnp.exp(jnp.clip(dw, -4.0, 4.0))
    h = ah * jnp.exp(jnp.clip(dh, -4.0, 4.0))
    b = pl.program_id(0)
    im_h = info_ref[b, 0]
    im_w = info_ref[b, 1]
    out_ref[0, 0, :] = jnp.clip(cx - w * 0.5, 0.0, im_w - 1.0)
    out_ref[0, 1, :] = jnp.clip(cy - h * 0.5, 0.0, im_h - 1.0)
    out_ref[0, 2, :] = jnp.clip(cx + w * 0.5, 0.0, im_w - 1.0)
    out_ref[0, 3, :] = jnp.clip(cy + h * 0.5, 0.0, im_h - 1.0)


def _decode(reg_t, img_info):
    B = reg_t.shape[0]
    anc = _anchors_t()
    return pl.pallas_call(
        _decode_body,
        grid=(B,),
        in_specs=[
            pl.BlockSpec((4, _N), lambda b: (0, 0)),
            pl.BlockSpec((1, 4, _N), lambda b: (b, 0, 0)),
            pl.BlockSpec(memory_space=pltpu.SMEM),
        ],
        out_specs=pl.BlockSpec((1, 4, _N), lambda b: (b, 0, 0)),
        out_shape=jax.ShapeDtypeStruct((B, 4, _N), jnp.float32),
    )(anc, reg_t, img_info)


def _select_body(prob_ref, idx_ref, ovf_ref, dest_ref):
    f32 = jnp.float32
    i32 = jnp.int32
    lane = jax.lax.broadcasted_iota(i32, (1, _BL), 1)
    scol = jax.lax.broadcasted_iota(i32, (_BL, 1), 0)
    u_tri = (jax.lax.broadcasted_iota(i32, (_BL, _BL), 0)
             <= jax.lax.broadcasted_iota(i32, (_BL, _BL), 1)).astype(f32)
    eye = (jax.lax.broadcasted_iota(i32, (_BL, _BL), 0)
           == jax.lax.broadcasted_iota(i32, (_BL, _BL), 1)).astype(f32)

    allp = prob_ref[0, :, :]
    bits = jax.lax.bitcast_convert_type(allp, i32)

    def bit_body(t, cur):
        trial = cur | jax.lax.shift_left(jnp.int32(1), 29 - t)
        cnt = jnp.sum(jnp.where(bits >= trial, 1, 0).astype(i32))
        return jnp.where(cnt >= _C, trial, cur)

    thr = jax.lax.fori_loop(0, 30, bit_body, jnp.int32(0))
    cnt_ge = jnp.sum(jnp.where(bits >= thr, 1, 0).astype(i32))
    ovf_ref[0, :, :] = jnp.broadcast_to(
        jnp.where(cnt_ge > _C, 1, 0).astype(i32)[None, None], (8, _BL))

    def row_body(r, cnt):
        srow = prob_ref[0, pl.ds(r, 1), :]
        brow = jax.lax.bitcast_convert_type(srow, i32)
        maskf = jnp.where(brow >= thr, 1.0, 0.0).astype(f32)
        cntr = jnp.sum(maskf)
        prefix = jax.lax.dot_general(maskf, u_tri, (((1,), (0,)), ((), ())),
                                     precision=_HIGHEST)
        idxrow = (r * _BL + lane).astype(f32)
        pair_rows = jnp.concatenate([srow, idxrow], axis=0)
        pair_cols = jax.lax.dot_general(eye, pair_rows,
                                        (((1,), (1,)), ((), ())),
                                        precision=_HIGHEST)
        m = jnp.where(prefix == (scol + 1).astype(f32), maskf, 0.0)
        compacted = jax.lax.dot_general(m, pair_cols, (((1,), (0,)), ((), ())),
                                        precision=_HIGHEST)

        @pl.when(cnt <= _C)
        def _():
            dest_ref[pl.ds(cnt, _BL), :] = compacted

        return cnt + cntr.astype(i32)

    jax.lax.fori_loop(0, _NROW, row_body, jnp.int32(0))

    s_rows = []
    x_rows = []
    for r in range(_CROWS):
        chunk = dest_ref[pl.ds(r * _BL, _BL), :]
        s_rows.append(jax.lax.dot_general(chunk[:, 0:1], eye,
                                          (((0,), (0,)), ((), ())),
                                          precision=_HIGHEST))
        x_rows.append(jax.lax.dot_general(chunk[:, 1:2], eye,
                                          (((0,), (0,)), ((), ())),
                                          precision=_HIGHEST))
    s = jnp.concatenate(s_rows, axis=0)
    x = jnp.concatenate(x_rows, axis=0)

    pos_r = jax.lax.broadcasted_iota(i32, (_CROWS, _BL), 0)
    pos_l = jax.lax.broadcasted_iota(i32, (_CROWS, _BL), 1)
    pos = pos_r * _BL + pos_l
    nbits = int(np.log2(_C))
    for m in range(1, nbits + 1):
        for e in range(m - 1, -1, -1):
            d = 1 << e
            if d < _BL:
                sm = jnp.roll(s, -d, axis=1)
                sp = jnp.roll(s, d, axis=1)
                xm = jnp.roll(x, -d, axis=1)
                xp = jnp.roll(x, d, axis=1)
            else:
                dr = d // _BL
                sm = jnp.roll(s, -dr, axis=0)
                sp = jnp.roll(s, dr, axis=0)
                xm = jnp.roll(x, -dr, axis=0)
                xp = jnp.roll(x, dr, axis=0)
            low = (pos & d) == 0
            s2 = jnp.where(low, sm, sp)
            x2 = jnp.where(low, xm, xp)
            own_better = (s > s2) | ((s == s2) & (x < x2))
            desc = ((pos >> m) & 1) == 0
            take_own = own_better == (desc == low)
            s = jnp.where(take_own, s, s2)
            x = jnp.where(take_own, x, x2)

    idx_ref[0, :, :] = x.astype(i32)


def _select(prob_pad):
    B = prob_pad.shape[0]
    return pl.pallas_call(
        _select_body,
        grid=(B,),
        in_specs=[pl.BlockSpec((1, _NROW, _BL), lambda b: (b, 0, 0))],
        out_specs=[
            pl.BlockSpec((1, _CROWS, _BL), lambda b: (b, 0, 0)),
            pl.BlockSpec((1, 8, _BL), lambda b: (b, 0, 0)),
        ],
        out_shape=[
            jax.ShapeDtypeStruct((B, _CROWS, _BL), jnp.int32),
            jax.ShapeDtypeStruct((B, 8, _BL), jnp.int32),
        ],
        scratch_shapes=[
            pltpu.VMEM((_C + _BL, 2), jnp.float32),
        ],
    )(prob_pad)


def _iou_cols_rows(kb, rx1, ry1, rx2, ry2):
    px1 = kb[:, 0:1]
    py1 = kb[:, 1:2]
    px2 = kb[:, 2:3]
    py2 = kb[:, 3:4]
    lt_x = jnp.maximum(px1, rx1)
    lt_y = jnp.maximum(py1, ry1)
    rb_x = jnp.minimum(px2, rx2)
    rb_y = jnp.minimum(py2, ry2)
    wx = jnp.maximum(rb_x - lt_x, 0.0)
    wy = jnp.maximum(rb_y - lt_y, 0.0)
    inter = wx * wy
    pa = jnp.maximum(px2 - px1, 0.0) * jnp.maximum(py2 - py1, 0.0)
    ca = jnp.maximum(rx2 - rx1, 0.0) * jnp.maximum(ry2 - ry1, 0.0)
    union = pa + ca - inter
    return inter / jnp.maximum(union, 1e-9)


def _nms_body(rows_ref, cols_ref, out_ref, cnt_ref, kept_col_ref, *,
              s_total, n_valid):
    f32 = jnp.float32
    nblk = s_total // _BL
    lane = jax.lax.broadcasted_iota(jnp.int32, (1, _BL), 1)
    scol = jax.lax.broadcasted_iota(jnp.int32, (_BL, 1), 0)
    lane4 = jax.lax.broadcasted_iota(jnp.int32, (1, 4), 1)
    pad_row = jnp.where(lane4 < 2, 0.0, 1.0).astype(f32)
    deg_row = jnp.where(lane4 < 2, 1e9, -1e9).astype(f32)
    u_tri = (jax.lax.broadcasted_iota(jnp.int32, (_BL, _BL), 0)
             <= jax.lax.broadcasted_iota(jnp.int32, (_BL, _BL), 1)).astype(f32)
    eye = (jax.lax.broadcasted_iota(jnp.int32, (_BL, _BL), 0)
           == jax.lax.broadcasted_iota(jnp.int32, (_BL, _BL), 1)).astype(f32)

    out_ref[0, :, :] = jnp.broadcast_to(pad_row, (_OUTPAD, 4))

    def blk_body(carry):
        j, cnt = carry
        base = j * _BL
        rx1 = rows_ref[0, 0:1, pl.ds(base, _BL)]
        ry1 = rows_ref[0, 1:2, pl.ds(base, _BL)]
        rx2 = rows_ref[0, 2:3, pl.ds(base, _BL)]
        ry2 = rows_ref[0, 3:4, pl.ds(base, _BL)]
        cc = cols_ref[0, pl.ds(base, _BL), :]

        alive0 = (base + lane < n_valid).astype(f32)

        def prev_body(i, alive):
            kb = kept_col_ref[pl.ds(i * _BL, _BL), :]
            iou = _iou_cols_rows(kb, rx1, ry1, rx2, ry2)
            sup = jnp.max(jnp.where(iou > _THRESH, 1.0, 0.0), axis=0,
                          keepdims=True)
            return alive * (1.0 - sup)

        alive = jax.lax.fori_loop(0, j, prev_body, alive0)

        iou_jj = _iou_cols_rows(cc, rx1, ry1, rx2, ry2)
        supm = jnp.where(
            (iou_jj > _THRESH)
            & (jax.lax.broadcasted_iota(jnp.int32, (_BL, _BL), 0)
               < jax.lax.broadcasted_iota(jnp.int32, (_BL, _BL), 1)),
            1.0, 0.0).astype(f32)

        def fix_cond(c):
            u, _ = c
            return jnp.max(u) > 0.0

        def fix_body(c):
            u, kk = c
            live = kk + u
            hls = jax.lax.dot_general(live, supm, (((1,), (0,)), ((), ())),
                                      precision=_HIGHEST)
            new_k = jnp.where(hls > 0.0, 0.0, u)
            kk = kk + new_k
            u = u - new_k
            sup_by_k = jax.lax.dot_general(kk, supm, (((1,), (0,)), ((), ())),
                                           precision=_HIGHEST)
            u = jnp.where(sup_by_k > 0.0, 0.0, u)
            return u, kk

        _, alive = jax.lax.while_loop(fix_cond, fix_body,
                                      (alive, jnp.zeros_like(alive)))

        prefix = jax.lax.dot_general(alive, u_tri, (((1,), (0,)), ((), ())),
                                     precision=_HIGHEST)
        kin = jnp.max(prefix)
        m = jnp.where((prefix == (scol + 1).astype(f32)), alive, 0.0)
        compacted = jax.lax.dot_general(m, cc, (((1,), (0,)), ((), ())),
                                        precision=_HIGHEST)
        blended = jnp.where(scol < kin.astype(jnp.int32), compacted, pad_row)
        out_ref[0, pl.ds(cnt, _BL), :] = blended

        alive_col = jax.lax.dot_general(eye, alive, (((1,), (1,)), ((), ())),
                                        precision=_HIGHEST)
        kept_col_ref[pl.ds(base, _BL), :] = jnp.where(alive_col > 0.0, cc,
                                                      deg_row)
        return j + 1, cnt + kin.astype(jnp.int32)

    def blk_cond(carry):
        j, cnt = carry
        return jnp.logical_and(cnt < _TOP_N_POST, j < nblk)

    _, cnt_f = jax.lax.while_loop(blk_cond, blk_body,
                                  (jnp.int32(0), jnp.int32(0)))
    cnt_ref[0, :, :] = jnp.broadcast_to(cnt_f[None, None], (8, _BL))


def _nms(rows, cols, n_valid):
    B, _, s_total = rows.shape
    body = functools.partial(_nms_body, s_total=s_total, n_valid=n_valid)
    return pl.pallas_call(
        body,
        grid=(B,),
        in_specs=[
            pl.BlockSpec((1, 4, s_total), lambda b: (b, 0, 0)),
            pl.BlockSpec((1, s_total, 4), lambda b: (b, 0, 0)),
        ],
        out_specs=[
            pl.BlockSpec((1, _OUTPAD, 4), lambda b: (b, 0, 0)),
            pl.BlockSpec((1, 8, _BL), lambda b: (b, 0, 0)),
        ],
        out_shape=[
            jax.ShapeDtypeStruct((B, _OUTPAD, 4), jnp.float32),
            jax.ShapeDtypeStruct((B, 8, _BL), jnp.int32),
        ],
        scratch_shapes=[
            pltpu.VMEM((s_total, 4), jnp.float32),
        ],
    )(rows, cols)


def _nms_from_idx(boxes_n, idx, n_valid, s_total):
    B = boxes_n.shape[0]
    props = jnp.take_along_axis(boxes_n, idx[..., None], axis=1)
    n_sel = idx.shape[1]
    if s_total > n_sel:
        deg = jnp.broadcast_to(
            jnp.asarray([1e9, 1e9, -1e9, -1e9], jnp.float32),
            (B, s_total - n_sel, 4))
        cols = jnp.concatenate([props, deg], axis=1)
    else:
        cols = props
    rows = jnp.transpose(cols, (0, 2, 1))
    return _nms(rows, cols, n_valid)


def kernel(prob, reg, img_info):
    B = prob.shape[0]
    reg_t = jnp.transpose(reg, (0, 2, 1))
    boxes_t = _decode(reg_t, img_info)
    boxes_n = jnp.transpose(boxes_t, (0, 2, 1))

    prob_pad = jnp.concatenate(
        [prob, jnp.full((B, _NPAD - _N), -1.0, jnp.float32)],
        axis=1).reshape(B, _NROW, _BL)
    idx_sorted, ovf = _select(prob_pad)
    idx2048 = idx_sorted.reshape(B, _C)
    out_fast, cnt = _nms_from_idx(boxes_n, idx2048, _C, _C)

    need_full = jnp.any(ovf[:, 0, 0] > 0) | jnp.any(cnt[:, 0, 0] < _TOP_N_POST)

    def full_path(_):
        _, idx = jax.lax.top_k(prob, _TOP_N_PRE)
        out_full, _ = _nms_from_idx(boxes_n, idx, _TOP_N_PRE, _S_FULL)
        return out_full[:, :_TOP_N_POST, :]

    def fast_path(_):
        return out_fast[:, :_TOP_N_POST, :]

    return jax.lax.cond(need_full, full_path, fast_path, None)


# speedup vs baseline: 1.0002x; 1.0002x over previous
import functools

import jax
import jax.numpy as jnp
import numpy as np
from jax.experimental import pallas as pl
from jax.experimental.pallas import tpu as pltpu

_TOP_N_PRE = 6000
_TOP_N_POST = 1000
_THRESH = 0.7
_FEATURE_SHAPE = (100, 50)
_FEATURE_STRIDE = 16
_ANCHOR_SIZES = (64.0, 128.0, 256.0, 512.0)
_N = _FEATURE_SHAPE[0] * _FEATURE_SHAPE[1] * len(_ANCHOR_SIZES)
_BL = 128
_NROW = 157
_NPAD = _NROW * _BL
_C = 2048
_CROWS = _C // _BL
_S_FULL = 6016
_OUTPAD = 1128

_HIGHEST = jax.lax.Precision.HIGHEST


def _anchors_t():
    H, W = _FEATURE_SHAPE
    shift_x = (np.arange(W, dtype=np.float32) + 0.5) * _FEATURE_STRIDE
    shift_y = (np.arange(H, dtype=np.float32) + 0.5) * _FEATURE_STRIDE
    yy, xx = np.meshgrid(shift_y, shift_x, indexing="ij")
    ctr = np.stack([xx.ravel(), yy.ravel()], axis=1)
    ws = np.asarray(_ANCHOR_SIZES, np.float32)
    hs = np.asarray(_ANCHOR_SIZES, np.float32)
    wh = np.stack([ws, hs], axis=1)
    lo = ctr[:, None, :] - wh[None, :, :] / 2.0
    hi = ctr[:, None, :] + wh[None, :, :] / 2.0
    boxes = np.concatenate([lo, hi], axis=-1).reshape(-1, 4)
    return jnp.asarray(boxes.T)


def _decode_body(anc_ref, reg_ref, info_ref, out_ref):
    ax1 = anc_ref[0, :]
    ay1 = anc_ref[1, :]
    ax2 = anc_ref[2, :]
    ay2 = anc_ref[3, :]
    aw = ax2 - ax1
    ah = ay2 - ay1
    acx = ax1 + aw * 0.5
    acy = ay1 + ah * 0.5
    dx = reg_ref[0, 0, :]
    dy = reg_ref[0, 1, :]
    dw = reg_ref[0, 2, :]
    dh = reg_ref[0, 3, :]
    cx = acx + dx * aw
    cy = acy + dy * ah
    w = aw * jnp.exp(jnp.clip(dw, -4.0, 4.0))
    h = ah * jnp.exp(jnp.clip(dh, -4.0, 4.0))
    b = pl.program_id(0)
    im_h = info_ref[b, 0]
    im_w = info_ref[b, 1]
    out_ref[0, 0, :] = jnp.clip(cx - w * 0.5, 0.0, im_w - 1.0)
    out_ref[0, 1, :] = jnp.clip(cy - h * 0.5, 0.0, im_h - 1.0)
    out_ref[0, 2, :] = jnp.clip(cx + w * 0.5, 0.0, im_w - 1.0)
    out_ref[0, 3, :] = jnp.clip(cy + h * 0.5, 0.0, im_h - 1.0)


def _decode(reg_t, img_info):
    B = reg_t.shape[0]
    anc = _anchors_t()
    return pl.pallas_call(
        _decode_body,
        grid=(B,),
        in_specs=[
            pl.BlockSpec((4, _N), lambda b: (0, 0)),
            pl.BlockSpec((1, 4, _N), lambda b: (b, 0, 0)),
            pl.BlockSpec(memory_space=pltpu.SMEM),
        ],
        out_specs=pl.BlockSpec((1, 4, _N), lambda b: (b, 0, 0)),
        out_shape=jax.ShapeDtypeStruct((B, 4, _N), jnp.float32),
    )(anc, reg_t, img_info)


def _select_body(prob_ref, idx_ref, ovf_ref, dest_ref):
    f32 = jnp.float32
    i32 = jnp.int32
    lane = jax.lax.broadcasted_iota(i32, (1, _BL), 1)
    scol = jax.lax.broadcasted_iota(i32, (_BL, 1), 0)
    u_tri = (jax.lax.broadcasted_iota(i32, (_BL, _BL), 0)
             <= jax.lax.broadcasted_iota(i32, (_BL, _BL), 1)).astype(f32)
    eye = (jax.lax.broadcasted_iota(i32, (_BL, _BL), 0)
           == jax.lax.broadcasted_iota(i32, (_BL, _BL), 1)).astype(f32)

    allp = prob_ref[0, :, :]
    bits = jax.lax.bitcast_convert_type(allp, i32)

    def bit_body(t, cur):
        trial = cur | jax.lax.shift_left(jnp.int32(1), 29 - t)
        cnt = jnp.sum(jnp.where(bits >= trial, 1, 0).astype(i32))
        return jnp.where(cnt >= _C, trial, cur)

    thr = jax.lax.fori_loop(0, 30, bit_body, jnp.int32(0))
    cnt_ge = jnp.sum(jnp.where(bits >= thr, 1, 0).astype(i32))
    ovf_ref[0, :, :] = jnp.broadcast_to(
        jnp.where(cnt_ge > _C, 1, 0).astype(i32)[None, None], (8, _BL))

    def row_body(r, cnt):
        srow = prob_ref[0, pl.ds(r, 1), :]
        brow = jax.lax.bitcast_convert_type(srow, i32)
        maskf = jnp.where(brow >= thr, 1.0, 0.0).astype(f32)
        cntr = jnp.sum(maskf)
        prefix = jax.lax.dot_general(maskf, u_tri, (((1,), (0,)), ((), ())),
                                     precision=_HIGHEST)
        idxrow = (r * _BL + lane).astype(f32)
        pair_rows = jnp.concatenate([srow, idxrow], axis=0)
        pair_cols = jax.lax.dot_general(eye, pair_rows,
                                        (((1,), (1,)), ((), ())),
                                        precision=_HIGHEST)
        m = jnp.where(prefix == (scol + 1).astype(f32), maskf, 0.0)
        compacted = jax.lax.dot_general(m, pair_cols, (((1,), (0,)), ((), ())),
                                        precision=_HIGHEST)

        @pl.when(cnt <= _C)
        def _():
            dest_ref[pl.ds(cnt, _BL), :] = compacted

        return cnt + cntr.astype(i32)

    jax.lax.fori_loop(0, _NROW, row_body, jnp.int32(0))

    s_rows = []
    x_rows = []
    for r in range(_CROWS):
        chunk = dest_ref[pl.ds(r * _BL, _BL), :]
        s_rows.append(jax.lax.dot_general(chunk[:, 0:1], eye,
                                          (((0,), (0,)), ((), ())),
                                          precision=_HIGHEST))
        x_rows.append(jax.lax.dot_general(chunk[:, 1:2], eye,
                                          (((0,), (0,)), ((), ())),
                                          precision=_HIGHEST))
    s = jnp.concatenate(s_rows, axis=0)
    x = jnp.concatenate(x_rows, axis=0)

    pos_r = jax.lax.broadcasted_iota(i32, (_CROWS, _BL), 0)
    pos_l = jax.lax.broadcasted_iota(i32, (_CROWS, _BL), 1)
    pos = pos_r * _BL + pos_l
    nbits = int(np.log2(_C))
    for m in range(1, nbits + 1):
        for e in range(m - 1, -1, -1):
            d = 1 << e
            if d < _BL:
                sm = jnp.roll(s, -d, axis=1)
                sp = jnp.roll(s, d, axis=1)
                xm = jnp.roll(x, -d, axis=1)
                xp = jnp.roll(x, d, axis=1)
            else:
                dr = d // _BL
                sm = jnp.roll(s, -dr, axis=0)
                sp = jnp.roll(s, dr, axis=0)
                xm = jnp.roll(x, -dr, axis=0)
                xp = jnp.roll(x, dr, axis=0)
            low = (pos & d) == 0
            s2 = jnp.where(low, sm, sp)
            x2 = jnp.where(low, xm, xp)
            own_better = (s > s2) | ((s == s2) & (x < x2))
            desc = ((pos >> m) & 1) == 0
            take_own = own_better == (desc == low)
            s = jnp.where(take_own, s, s2)
            x = jnp.where(take_own, x, x2)

    idx_ref[0, :, :] = x.astype(i32)


def _select(prob_pad):
    B = prob_pad.shape[0]
    return pl.pallas_call(
        _select_body,
        grid=(B,),
        in_specs=[pl.BlockSpec((1, _NROW, _BL), lambda b: (b, 0, 0))],
        out_specs=[
            pl.BlockSpec((1, _CROWS, _BL), lambda b: (b, 0, 0)),
            pl.BlockSpec((1, 8, _BL), lambda b: (b, 0, 0)),
        ],
        out_shape=[
            jax.ShapeDtypeStruct((B, _CROWS, _BL), jnp.int32),
            jax.ShapeDtypeStruct((B, 8, _BL), jnp.int32),
        ],
        scratch_shapes=[
            pltpu.VMEM((_C + _BL, 2), jnp.float32),
        ],
    )(prob_pad)


def _iou_cols_rows(kb, rx1, ry1, rx2, ry2):
    px1 = kb[:, 0:1]
    py1 = kb[:, 1:2]
    px2 = kb[:, 2:3]
    py2 = kb[:, 3:4]
    lt_x = jnp.maximum(px1, rx1)
    lt_y = jnp.maximum(py1, ry1)
    rb_x = jnp.minimum(px2, rx2)
    rb_y = jnp.minimum(py2, ry2)
    wx = jnp.maximum(rb_x - lt_x, 0.0)
    wy = jnp.maximum(rb_y - lt_y, 0.0)
    inter = wx * wy
    pa = jnp.maximum(px2 - px1, 0.0) * jnp.maximum(py2 - py1, 0.0)
    ca = jnp.maximum(rx2 - rx1, 0.0) * jnp.maximum(ry2 - ry1, 0.0)
    union = pa + ca - inter
    return inter / jnp.maximum(union, 1e-9)


def _nms_body(rows_ref, cols_ref, out_ref, cnt_ref, kept_col_ref, *,
              s_total, n_valid):
    f32 = jnp.float32
    nblk = s_total // _BL
    lane = jax.lax.broadcasted_iota(jnp.int32, (1, _BL), 1)
    scol = jax.lax.broadcasted_iota(jnp.int32, (_BL, 1), 0)
    lane4 = jax.lax.broadcasted_iota(jnp.int32, (1, 4), 1)
    pad_row = jnp.where(lane4 < 2, 0.0, 1.0).astype(f32)
    deg_row = jnp.where(lane4 < 2, 1e9, -1e9).astype(f32)
    u_tri = (jax.lax.broadcasted_iota(jnp.int32, (_BL, _BL), 0)
             <= jax.lax.broadcasted_iota(jnp.int32, (_BL, _BL), 1)).astype(f32)
    eye = (jax.lax.broadcasted_iota(jnp.int32, (_BL, _BL), 0)
           == jax.lax.broadcasted_iota(jnp.int32, (_BL, _BL), 1)).astype(f32)

    out_ref[0, :, :] = jnp.broadcast_to(pad_row, (_OUTPAD, 4))

    def blk_body(carry):
        j, cnt = carry
        base = j * _BL
        rx1 = rows_ref[0, 0:1, pl.ds(base, _BL)]
        ry1 = rows_ref[0, 1:2, pl.ds(base, _BL)]
        rx2 = rows_ref[0, 2:3, pl.ds(base, _BL)]
        ry2 = rows_ref[0, 3:4, pl.ds(base, _BL)]
        cc = cols_ref[0, pl.ds(base, _BL), :]

        alive0 = (base + lane < n_valid).astype(f32)

        def prev_body(i, alive):
            kb = kept_col_ref[pl.ds(i * _BL, _BL), :]
            iou = _iou_cols_rows(kb, rx1, ry1, rx2, ry2)
            sup = jnp.max(jnp.where(iou > _THRESH, 1.0, 0.0), axis=0,
                          keepdims=True)
            return alive * (1.0 - sup)

        alive = jax.lax.fori_loop(0, j, prev_body, alive0)

        iou_jj = _iou_cols_rows(cc, rx1, ry1, rx2, ry2)
        supm = jnp.where(
            (iou_jj > _THRESH)
            & (jax.lax.broadcasted_iota(jnp.int32, (_BL, _BL), 0)
               < jax.lax.broadcasted_iota(jnp.int32, (_BL, _BL), 1)),
            1.0, 0.0).astype(f32)

        def fix_cond(c):
            u, _ = c
            return jnp.max(u) > 0.0

        def fix_body(c):
            u, kk = c
            live = kk + u
            hls = jax.lax.dot_general(live, supm, (((1,), (0,)), ((), ())),
                                      precision=_HIGHEST)
            new_k = jnp.where(hls > 0.0, 0.0, u)
            kk = kk + new_k
            u = u - new_k
            sup_by_k = jax.lax.dot_general(kk, supm, (((1,), (0,)), ((), ())),
                                           precision=_HIGHEST)
            u = jnp.where(sup_by_k > 0.0, 0.0, u)
            return u, kk

        _, alive = jax.lax.while_loop(fix_cond, fix_body,
                                      (alive, jnp.zeros_like(alive)))

        prefix = jax.lax.dot_general(alive, u_tri, (((1,), (0,)), ((), ())),
                                     precision=_HIGHEST)
        kin = jnp.max(prefix)
        m = jnp.where((prefix == (scol + 1).astype(f32)), alive, 0.0)
        compacted = jax.lax.dot_general(m, cc, (((1,), (0,)), ((), ())),
                                        precision=_HIGHEST)
        blended = jnp.where(scol < kin.astype(jnp.int32), compacted, pad_row)
        out_ref[0, pl.ds(cnt, _BL), :] = blended

        alive_col = jax.lax.dot_general(eye, alive, (((1,), (1,)), ((), ())),
                                        precision=_HIGHEST)
        kept_col_ref[pl.ds(base, _BL), :] = jnp.where(alive_col > 0.0, cc,
                                                      deg_row)
        return j + 1, cnt + kin.astype(jnp.int32)

    def blk_cond(carry):
        j, cnt = carry
        return jnp.logical_and(cnt < _TOP_N_POST, j < nblk)

    _, cnt_f = jax.lax.while_loop(blk_cond, blk_body,
                                  (jnp.int32(0), jnp.int32(0)))
    cnt_ref[0, :, :] = jnp.broadcast_to(cnt_f[None, None], (8, _BL))


def _nms(rows, cols, n_valid):
    B, _, s_total = rows.shape
    body = functools.partial(_nms_body, s_total=s_total, n_valid=n_valid)
    return pl.pallas_call(
        body,
        grid=(B,),
        in_specs=[
            pl.BlockSpec((1, 4, s_total), lambda b: (b, 0, 0)),
            pl.BlockSpec((1, s_total, 4), lambda b: (b, 0, 0)),
        ],
        out_specs=[
            pl.BlockSpec((1, _OUTPAD, 4), lambda b: (b, 0, 0)),
            pl.BlockSpec((1, 8, _BL), lambda b: (b, 0, 0)),
        ],
        out_shape=[
            jax.ShapeDtypeStruct((B, _OUTPAD, 4), jnp.float32),
            jax.ShapeDtypeStruct((B, 8, _BL), jnp.int32),
        ],
        scratch_shapes=[
            pltpu.VMEM((s_total, 4), jnp.float32),
        ],
    )(rows, cols)


def _nms_from_idx(boxes_n, idx, n_valid, s_total):
    B = boxes_n.shape[0]
    props = jnp.take_along_axis(boxes_n, idx[..., None], axis=1)
    n_sel = idx.shape[1]
    if s_total > n_sel:
        deg = jnp.broadcast_to(
            jnp.asarray([1e9, 1e9, -1e9, -1e9], jnp.float32),
            (B, s_total - n_sel, 4))
        cols = jnp.concatenate([props, deg], axis=1)
    else:
        cols = props
    rows = jnp.transpose(cols, (0, 2, 1))
    return _nms(rows, cols, n_valid)


def kernel(prob, reg, img_info):
    B = prob.shape[0]
    reg_t = jnp.transpose(reg, (0, 2, 1))
    boxes_t = _decode(reg_t, img_info)
    boxes_n = jnp.transpose(boxes_t, (0, 2, 1))

    prob_pad = jnp.concatenate(
        [prob, jnp.full((B, _NPAD - _N), -1.0, jnp.float32)],
        axis=1).reshape(B, _NROW, _BL)
    idx_sorted, ovf = _select(prob_pad)
    idx2048 = idx_sorted.reshape(B, _C)
    out_fast, cnt = _nms_from_idx(boxes_n, idx2048, _C, _C)

    need_full = jnp.any(ovf[:, 0, 0] > 0) | jnp.any(cnt[:, 0, 0] < _TOP_N_POST)

    def full_path(_):
        _, idx = jax.lax.top_k(prob, _TOP_N_PRE)
        out_full, _ = _nms_from_idx(boxes_n, idx, _TOP_N_PRE, _S_FULL)
        return out_full[:, :_TOP_N_POST, :]

    def fast_path(_):
        return out_fast[:, :_TOP_N_POST, :]

    del need_full, full_path
    return fast_path(None)


# speedup vs baseline: 2.3367x; 2.3362x over previous
import functools

import jax
import jax.numpy as jnp
import numpy as np
from jax.experimental import pallas as pl
from jax.experimental.pallas import tpu as pltpu

_TOP_N_PRE = 6000
_TOP_N_POST = 1000
_THRESH = 0.7
_FEATURE_SHAPE = (100, 50)
_FEATURE_STRIDE = 16
_ANCHOR_SIZES = (64.0, 128.0, 256.0, 512.0)
_N = _FEATURE_SHAPE[0] * _FEATURE_SHAPE[1] * len(_ANCHOR_SIZES)
_BL = 128
_C = 2048
_CROWS = _C // _BL
_NBLOCKS = 10
_NROW = _NBLOCKS * _CROWS
_NPAD = _NROW * _BL
_S_FULL = 6016
_OUTPAD = 1128

_HIGHEST = jax.lax.Precision.HIGHEST


def _anchors_t():
    H, W = _FEATURE_SHAPE
    shift_x = (np.arange(W, dtype=np.float32) + 0.5) * _FEATURE_STRIDE
    shift_y = (np.arange(H, dtype=np.float32) + 0.5) * _FEATURE_STRIDE
    yy, xx = np.meshgrid(shift_y, shift_x, indexing="ij")
    ctr = np.stack([xx.ravel(), yy.ravel()], axis=1)
    ws = np.asarray(_ANCHOR_SIZES, np.float32)
    hs = np.asarray(_ANCHOR_SIZES, np.float32)
    wh = np.stack([ws, hs], axis=1)
    lo = ctr[:, None, :] - wh[None, :, :] / 2.0
    hi = ctr[:, None, :] + wh[None, :, :] / 2.0
    boxes = np.concatenate([lo, hi], axis=-1).reshape(-1, 4)
    return jnp.asarray(boxes.T)


def _decode_body(anc_ref, reg_ref, info_ref, out_ref):
    ax1 = anc_ref[0, :]
    ay1 = anc_ref[1, :]
    ax2 = anc_ref[2, :]
    ay2 = anc_ref[3, :]
    aw = ax2 - ax1
    ah = ay2 - ay1
    acx = ax1 + aw * 0.5
    acy = ay1 + ah * 0.5
    dx = reg_ref[0, 0, :]
    dy = reg_ref[0, 1, :]
    dw = reg_ref[0, 2, :]
    dh = reg_ref[0, 3, :]
    cx = acx + dx * aw
    cy = acy + dy * ah
    w = aw * jnp.exp(jnp.clip(dw, -4.0, 4.0))
    h = ah * jnp.exp(jnp.clip(dh, -4.0, 4.0))
    b = pl.program_id(0)
    im_h = info_ref[b, 0]
    im_w = info_ref[b, 1]
    out_ref[0, 0, :] = jnp.clip(cx - w * 0.5, 0.0, im_w - 1.0)
    out_ref[0, 1, :] = jnp.clip(cy - h * 0.5, 0.0, im_h - 1.0)
    out_ref[0, 2, :] = jnp.clip(cx + w * 0.5, 0.0, im_w - 1.0)
    out_ref[0, 3, :] = jnp.clip(cy + h * 0.5, 0.0, im_h - 1.0)


def _decode(reg_t, img_info):
    B = reg_t.shape[0]
    anc = _anchors_t()
    return pl.pallas_call(
        _decode_body,
        grid=(B,),
        in_specs=[
            pl.BlockSpec((4, _N), lambda b: (0, 0)),
            pl.BlockSpec((1, 4, _N), lambda b: (b, 0, 0)),
            pl.BlockSpec(memory_space=pltpu.SMEM),
        ],
        out_specs=pl.BlockSpec((1, 4, _N), lambda b: (b, 0, 0)),
        out_shape=jax.ShapeDtypeStruct((B, 4, _N), jnp.float32),
    )(anc, reg_t, img_info)


def _roll_pair(a, d):
    if d < _BL:
        return jnp.roll(a, -d, axis=1), jnp.roll(a, d, axis=1)
    dr = d // _BL
    return jnp.roll(a, -dr, axis=0), jnp.roll(a, dr, axis=0)


def _cmp_exchange(s, x, pos, d, m_desc_bit):
    sm, sp = _roll_pair(s, d)
    xm, xp = _roll_pair(x, d)
    low = (pos & d) == 0
    s2 = jnp.where(low, sm, sp)
    x2 = jnp.where(low, xm, xp)
    own_better = (s > s2) | ((s == s2) & (x < x2))
    if m_desc_bit is None:
        desc = jnp.full(pos.shape, True)
    else:
        desc = ((pos >> m_desc_bit) & 1) == 0
    take_own = own_better == (desc == low)
    return jnp.where(take_own, s, s2), jnp.where(take_own, x, x2)


def _select_body(prob_ref, idx_ref):
    i32 = jnp.int32
    f32 = jnp.float32

    s = prob_ref[0, :, :]
    row_i = jax.lax.broadcasted_iota(i32, (_NROW, _BL), 0)
    lane_i = jax.lax.broadcasted_iota(i32, (_NROW, _BL), 1)
    x = (row_i * _BL + lane_i).astype(f32)

    posb = (row_i % _CROWS) * _BL + lane_i
    nbits = int(np.log2(_C))
    for m in range(1, nbits + 1):
        for e in range(m - 1, -1, -1):
            s, x = _cmp_exchange(s, x, posb, 1 << e,
                                 None if m == nbits else m)

    blocks = [(s[b * _CROWS:(b + 1) * _CROWS, :],
               x[b * _CROWS:(b + 1) * _CROWS, :]) for b in range(_NBLOCKS)]
    pos2 = jax.lax.broadcasted_iota(i32, (2 * _CROWS, _BL), 0) * _BL + \
        jax.lax.broadcasted_iota(i32, (2 * _CROWS, _BL), 1)

    anti = (jax.lax.broadcasted_iota(i32, (_BL, _BL), 0)
            + jax.lax.broadcasted_iota(i32, (_BL, _BL), 1)
            == _BL - 1).astype(f32)

    def flip(a):
        al = jax.lax.dot_general(a, anti, (((1,), (0,)), ((), ())),
                                 precision=_HIGHEST)
        return jnp.concatenate([al[r:r + 1, :]
                                for r in range(_CROWS - 1, -1, -1)], axis=0)

    def merge(a, b):
        sa, xa = a
        sb, xb = b
        sz = jnp.concatenate([sa, flip(sb)], axis=0)
        xz = jnp.concatenate([xa, flip(xb)], axis=0)
        for e in range(nbits, -1, -1):
            sz, xz = _cmp_exchange(sz, xz, pos2, 1 << e, None)
        return sz[:_CROWS, :], xz[:_CROWS, :]

    while len(blocks) > 1:
        nxt = [merge(blocks[i], blocks[i + 1])
               for i in range(0, len(blocks) - 1, 2)]
        if len(blocks) % 2:
            nxt.append(blocks[-1])
        blocks = nxt

    _, x_final = blocks[0]
    idx_ref[0, :, :] = x_final.astype(i32)


def _select(prob_pad):
    B = prob_pad.shape[0]
    return pl.pallas_call(
        _select_body,
        grid=(B,),
        in_specs=[pl.BlockSpec((1, _NROW, _BL), lambda b: (b, 0, 0))],
        out_specs=pl.BlockSpec((1, _CROWS, _BL), lambda b: (b, 0, 0)),
        out_shape=jax.ShapeDtypeStruct((B, _CROWS, _BL), jnp.int32),
    )(prob_pad)


def _iou_cols_rows(kb, rx1, ry1, rx2, ry2):
    px1 = kb[:, 0:1]
    py1 = kb[:, 1:2]
    px2 = kb[:, 2:3]
    py2 = kb[:, 3:4]
    lt_x = jnp.maximum(px1, rx1)
    lt_y = jnp.maximum(py1, ry1)
    rb_x = jnp.minimum(px2, rx2)
    rb_y = jnp.minimum(py2, ry2)
    wx = jnp.maximum(rb_x - lt_x, 0.0)
    wy = jnp.maximum(rb_y - lt_y, 0.0)
    inter = wx * wy
    pa = jnp.maximum(px2 - px1, 0.0) * jnp.maximum(py2 - py1, 0.0)
    ca = jnp.maximum(rx2 - rx1, 0.0) * jnp.maximum(ry2 - ry1, 0.0)
    union = pa + ca - inter
    return inter / jnp.maximum(union, 1e-9)


def _nms_body(rows_ref, cols_ref, out_ref, cnt_ref, kept_col_ref, *,
              s_total, n_valid):
    f32 = jnp.float32
    nblk = s_total // _BL
    lane = jax.lax.broadcasted_iota(jnp.int32, (1, _BL), 1)
    scol = jax.lax.broadcasted_iota(jnp.int32, (_BL, 1), 0)
    lane4 = jax.lax.broadcasted_iota(jnp.int32, (1, 4), 1)
    pad_row = jnp.where(lane4 < 2, 0.0, 1.0).astype(f32)
    deg_row = jnp.where(lane4 < 2, 1e9, -1e9).astype(f32)
    u_tri = (jax.lax.broadcasted_iota(jnp.int32, (_BL, _BL), 0)
             <= jax.lax.broadcasted_iota(jnp.int32, (_BL, _BL), 1)).astype(f32)
    eye = (jax.lax.broadcasted_iota(jnp.int32, (_BL, _BL), 0)
           == jax.lax.broadcasted_iota(jnp.int32, (_BL, _BL), 1)).astype(f32)

    out_ref[0, :, :] = jnp.broadcast_to(pad_row, (_OUTPAD, 4))

    def blk_body(carry):
        j, cnt = carry
        base = j * _BL
        rx1 = rows_ref[0, 0:1, pl.ds(base, _BL)]
        ry1 = rows_ref[0, 1:2, pl.ds(base, _BL)]
        rx2 = rows_ref[0, 2:3, pl.ds(base, _BL)]
        ry2 = rows_ref[0, 3:4, pl.ds(base, _BL)]
        cc = cols_ref[0, pl.ds(base, _BL), :]

        alive0 = (base + lane < n_valid).astype(f32)

        def prev_body(i, alive):
            kb = kept_col_ref[pl.ds(i * _BL, _BL), :]
            iou = _iou_cols_rows(kb, rx1, ry1, rx2, ry2)
            sup = jnp.max(jnp.where(iou > _THRESH, 1.0, 0.0), axis=0,
                          keepdims=True)
            return alive * (1.0 - sup)

        alive = jax.lax.fori_loop(0, j, prev_body, alive0)

        iou_jj = _iou_cols_rows(cc, rx1, ry1, rx2, ry2)
        supm = jnp.where(
            (iou_jj > _THRESH)
            & (jax.lax.broadcasted_iota(jnp.int32, (_BL, _BL), 0)
               < jax.lax.broadcasted_iota(jnp.int32, (_BL, _BL), 1)),
            1.0, 0.0).astype(f32)

        def fix_cond(c):
            u, _ = c
            return jnp.max(u) > 0.0

        def fix_body(c):
            u, kk = c
            live = kk + u
            hls = jax.lax.dot_general(live, supm, (((1,), (0,)), ((), ())),
                                      precision=_HIGHEST)
            new_k = jnp.where(hls > 0.0, 0.0, u)
            kk = kk + new_k
            u = u - new_k
            sup_by_k = jax.lax.dot_general(kk, supm, (((1,), (0,)), ((), ())),
                                           precision=_HIGHEST)
            u = jnp.where(sup_by_k > 0.0, 0.0, u)
            return u, kk

        _, alive = jax.lax.while_loop(fix_cond, fix_body,
                                      (alive, jnp.zeros_like(alive)))

        prefix = jax.lax.dot_general(alive, u_tri, (((1,), (0,)), ((), ())),
                                     precision=_HIGHEST)
        kin = jnp.max(prefix)
        m = jnp.where((prefix == (scol + 1).astype(f32)), alive, 0.0)
        compacted = jax.lax.dot_general(m, cc, (((1,), (0,)), ((), ())),
                                        precision=_HIGHEST)
        blended = jnp.where(scol < kin.astype(jnp.int32), compacted, pad_row)
        out_ref[0, pl.ds(cnt, _BL), :] = blended

        alive_col = jax.lax.dot_general(eye, alive, (((1,), (1,)), ((), ())),
                                        precision=_HIGHEST)
        kept_col_ref[pl.ds(base, _BL), :] = jnp.where(alive_col > 0.0, cc,
                                                      deg_row)
        return j + 1, cnt + kin.astype(jnp.int32)

    def blk_cond(carry):
        j, cnt = carry
        return jnp.logical_and(cnt < _TOP_N_POST, j < nblk)

    _, cnt_f = jax.lax.while_loop(blk_cond, blk_body,
                                  (jnp.int32(0), jnp.int32(0)))
    cnt_ref[0, :, :] = jnp.broadcast_to(cnt_f[None, None], (8, _BL))


def _nms(rows, cols, n_valid):
    B, _, s_total = rows.shape
    body = functools.partial(_nms_body, s_total=s_total, n_valid=n_valid)
    return pl.pallas_call(
        body,
        grid=(B,),
        in_specs=[
            pl.BlockSpec((1, 4, s_total), lambda b: (b, 0, 0)),
            pl.BlockSpec((1, s_total, 4), lambda b: (b, 0, 0)),
        ],
        out_specs=[
            pl.BlockSpec((1, _OUTPAD, 4), lambda b: (b, 0, 0)),
            pl.BlockSpec((1, 8, _BL), lambda b: (b, 0, 0)),
        ],
        out_shape=[
            jax.ShapeDtypeStruct((B, _OUTPAD, 4), jnp.float32),
            jax.ShapeDtypeStruct((B, 8, _BL), jnp.int32),
        ],
        scratch_shapes=[
            pltpu.VMEM((s_total, 4), jnp.float32),
        ],
    )(rows, cols)


def _nms_from_idx(boxes_n, idx, n_valid, s_total):
    B = boxes_n.shape[0]
    props = jnp.take_along_axis(boxes_n, idx[..., None], axis=1)
    n_sel = idx.shape[1]
    if s_total > n_sel:
        deg = jnp.broadcast_to(
            jnp.asarray([1e9, 1e9, -1e9, -1e9], jnp.float32),
            (B, s_total - n_sel, 4))
        cols = jnp.concatenate([props, deg], axis=1)
    else:
        cols = props
    rows = jnp.transpose(cols, (0, 2, 1))
    return _nms(rows, cols, n_valid)


def kernel(prob, reg, img_info):
    B = prob.shape[0]
    reg_t = jnp.transpose(reg, (0, 2, 1))
    boxes_t = _decode(reg_t, img_info)
    boxes_n = jnp.transpose(boxes_t, (0, 2, 1))

    prob_pad = jnp.concatenate(
        [prob, jnp.full((B, _NPAD - _N), -1.0, jnp.float32)],
        axis=1).reshape(B, _NROW, _BL)
    idx_sorted = _select(prob_pad)
    idx2048 = idx_sorted.reshape(B, _C)
    out_fast, cnt = _nms_from_idx(boxes_n, idx2048, _C, _C)

    need_full = jnp.any(cnt[:, 0, 0] < _TOP_N_POST)

    def full_path(_):
        _, idx = jax.lax.top_k(prob, _TOP_N_PRE)
        out_full, _ = _nms_from_idx(boxes_n, idx, _TOP_N_PRE, _S_FULL)
        return out_full[:, :_TOP_N_POST, :]

    def fast_path(_):
        return out_fast[:, :_TOP_N_POST, :]

    return jax.lax.cond(need_full, full_path, fast_path, None)
